# baseline (speedup 1.0000x reference)
import jax
import jax.numpy as jnp
from jax.experimental import pallas as pl

N = 1024
C = 64
KQ = 16
KN = 8
BN = 512
_INV_S = float(1.0 / (1.0 + 1e-5) ** 0.5)
_PREC = jax.lax.Precision.HIGHEST


def _dotg_bf16(a, b):
    return jax.lax.dot_general(a.astype(jnp.bfloat16), b.astype(jnp.bfloat16),
                               (((1,), (1,)), ((), ())),
                               preferred_element_type=jnp.float32)


def _mm_bf16(a, w):
    return jax.lax.dot_general(a.astype(jnp.bfloat16), w.astype(jnp.bfloat16),
                               (((1,), (0,)), ((), ())),
                               preferred_element_type=jnp.float32)


def _hilo(tbl):
    hi = tbl.astype(jnp.bfloat16)
    lo = (tbl - hi.astype(jnp.float32)).astype(jnp.bfloat16)
    return hi, lo


def _gather_mm(oh, hi, lo):
    ohb = oh.astype(jnp.bfloat16)
    g = jax.lax.dot_general(ohb, hi, (((1,), (0,)), ((), ())),
                            preferred_element_type=jnp.float32)
    g += jax.lax.dot_general(ohb, lo, (((1,), (0,)), ((), ())),
                             preferred_element_type=jnp.float32)
    return g


def _row_sumsq(x):
    ones = jnp.ones((1, x.shape[1]), jnp.float32)
    return jax.lax.dot_general(ones, x * x, (((1,), (1,)), ((), ())),
                               preferred_element_type=jnp.float32,
                               precision=_PREC)


def _apply(x, w, b):
    return jnp.maximum(_mm_bf16(x, w) + b, 0.0)


def _argmin_onehot(d, iota):
    m = jnp.min(d, axis=1, keepdims=True)
    idx = jnp.min(jnp.where(d == m, iota, d.shape[1]), axis=1, keepdims=True)
    oh = (iota == idx)
    return oh, idx


def _nbr_kernel(xyz_ref, xyzf_ref, featsf_ref, w0, b0, w1, b1, w2, b2,
                nbr_ref, idx8_ref):
    xyz = xyz_ref[0]
    xyzf = xyzf_ref[0]
    featsf = featsf_ref[0]
    sq = jnp.sum(xyz * xyz, axis=1, keepdims=True)
    sq_row = _row_sumsq(xyzf)
    d = sq + sq_row - 2.0 * _dotg_bf16(xyz, xyzf)
    iota = jax.lax.broadcasted_iota(jnp.int32, (BN, N), 1)
    hi, lo = _hilo(jnp.concatenate([xyzf, featsf], axis=1))

    gath = []
    idx_cols = []
    for _ in range(KN):
        oh, idx = _argmin_onehot(d, iota)
        gath.append(_gather_mm(oh, hi, lo))
        idx_cols.append(idx)
        d = jnp.where(oh, 1e30, d)
    idx8_ref[0] = jnp.concatenate(idx_cols, axis=1)

    rows = []
    kfeats = []
    for g in gath:
        kxyz = g[:, :3]
        kf = g[:, 3:67]
        diff = kxyz - xyz
        dist = jnp.sqrt(jnp.sum(diff * diff, axis=1, keepdims=True))
        rows.append(jnp.concatenate([kf, diff, dist], axis=1)[None])
        kfeats.append(kf[None])
    x = jnp.concatenate(rows, axis=0).reshape(KN * BN, C + 4)
    x = _apply(x, w0[...], b0[...])
    x = _apply(x, w1[...], b1[...])
    x = _apply(x, w2[...], b2[...])
    m = jnp.max(x.reshape(KN, BN, C), axis=2)
    w = jax.nn.softmax(m, axis=0)
    kf = jnp.concatenate(kfeats, axis=0)
    nbr_ref[0] = jnp.sum(kf * w[:, :, None], axis=0)


def _stats_kernel(wp_ref, fp_ref, snbr_ref, dnbr_ref, out_ref):
    wp = wp_ref[0]
    fp = fp_ref[0]
    snbr = snbr_ref[0]
    dnbr = dnbr_ref[0]
    nd = jnp.sqrt(jnp.sum(fp * fp, axis=1, keepdims=True))
    ns_row = jnp.sqrt(_row_sumsq(wp))
    cosTT = _dotg_bf16(fp, wp) / (nd * ns_row + 1e-6)
    recA = 1.0 / (jnp.max(cosTT, axis=1, keepdims=True) + 1e-6)
    ndn = jnp.sqrt(jnp.sum(dnbr * dnbr, axis=1, keepdims=True))
    nsn_row = jnp.sqrt(_row_sumsq(snbr))
    cosNT = _dotg_bf16(dnbr, snbr) / (ndn * nsn_row + 1e-6)
    recAn = 1.0 / (jnp.max(cosNT, axis=1, keepdims=True) + 1e-6)
    out_ref[0] = jnp.concatenate([recA, recAn], axis=1)


def _cross_kernel(wxyz_ref, fxyz_ref, wp_ref, fp_ref, snbr_ref, dnbr_ref,
                  recs_ref, *refs):
    (c1w0, c1b0, c1w1, c1b1, c1w2, c1b2,
     m1w0, m1b0, m1w1, m1b1, m1w2, m1b2,
     pew, peb, m2w0, m2b0, m2w1, m2b1,
     out_ref) = refs
    wxyz = wxyz_ref[0]
    fxyz = fxyz_ref[0]
    wp = wp_ref[0]
    fp = fp_ref[0]
    snbr = snbr_ref[0]
    dnbr = dnbr_ref[0]
    recs = recs_ref[0]

    sqw = jnp.sum(wxyz * wxyz, axis=1, keepdims=True)
    sqf_row = _row_sumsq(fxyz)
    d = sqw + sqf_row - 2.0 * _dotg_bf16(wxyz, fxyz)

    ns = jnp.sqrt(jnp.sum(wp * wp, axis=1, keepdims=True))
    nd_row = jnp.sqrt(_row_sumsq(fp))
    cosT = _dotg_bf16(wp, fp) / (ns * nd_row + 1e-6)
    recR = 1.0 / (jnp.max(cosT, axis=1, keepdims=True) + 1e-6)

    nsn = jnp.sqrt(jnp.sum(snbr * snbr, axis=1, keepdims=True))
    ndn_row = jnp.sqrt(_row_sumsq(dnbr))
    cosN = _dotg_bf16(snbr, dnbr) / (nsn * ndn_row + 1e-6)
    recRn = 1.0 / (jnp.max(cosN, axis=1, keepdims=True) + 1e-6)

    iota = jax.lax.broadcasted_iota(jnp.int32, (BN, N), 1)
    hi, lo = _hilo(jnp.concatenate([fxyz, fp, recs], axis=1))

    dsts = []
    geoms = []
    cos4s = []
    for k in range(KQ):
        oh, _ = _argmin_onehot(d, iota)
        g = _gather_mm(oh, hi, lo)
        exCos = jnp.sum(jnp.where(oh, cosT, 0.0), axis=1, keepdims=True)
        exA = exCos * g[:, 67:68]
        exR = exCos * recR
        exCosN = jnp.sum(jnp.where(oh, cosN, 0.0), axis=1, keepdims=True)
        exAn = exCosN * g[:, 68:69]
        exRn = exCosN * recRn
        gxyz = g[:, :3]
        rela = gxyz - wxyz
        dist = jnp.sqrt(jnp.sum(rela * rela, axis=1, keepdims=True))
        dsts.append(g[:, 3:67][None])
        geoms.append(jnp.concatenate([rela, dist, wxyz, gxyz], axis=1)[None])
        cos4s.append(jnp.concatenate([exA, exR, exAn, exRn], axis=1)[None])
        d = jnp.where(oh, 1e30, d)

    rows = KQ * BN
    dst = jnp.concatenate(dsts, axis=0).reshape(rows, C)
    geom = jnp.concatenate(geoms, axis=0).reshape(rows, 10)
    cos4 = jnp.concatenate(cos4s, axis=0).reshape(rows, 4)
    srcf = jnp.broadcast_to(wp[None], (KQ, BN, C)).reshape(rows, C)

    x = jnp.concatenate([srcf, dst, geom, cos4], axis=1)
    x = _apply(x, c1w0[...], c1b0[...])
    x = _apply(x, c1w1[...], c1b1[...])
    x = _apply(x, c1w2[...], c1b2[...])

    pi = jnp.concatenate([srcf, dst, x, geom], axis=1)
    pi = _apply(pi, m1w0[...], m1b0[...])
    pi = _apply(pi, m1w1[...], m1b1[...])
    pi = _apply(pi, m1w2[...], m1b2[...])

    enc = _apply(geom, pew[...], peb[...])
    pc = jnp.concatenate([enc, pi], axis=1)
    pc = _apply(pc, m2w0[...], m2b0[...])
    pc = _apply(pc, m2w1[...], m2b1[...])

    wq = jax.nn.softmax(pc.reshape(KQ, BN, C), axis=0)
    out_ref[0] = jnp.sum(wq * pi.reshape(KQ, BN, C), axis=0)


def _pc_kernel(wxyz_ref, wxyzf_ref, wp_ref, piff_ref, idx8_ref, *refs):
    (pcw, pcb, nw0, nb0, nw1, nb1, out_ref) = refs
    wxyz = wxyz_ref[0]
    wxyzf = wxyzf_ref[0]
    wp = wp_ref[0]
    piff = piff_ref[0]
    idx8 = idx8_ref[0]
    hi, lo = _hilo(jnp.concatenate([wxyzf, piff], axis=1))
    iota = jax.lax.broadcasted_iota(jnp.int32, (BN, N), 1)

    geoms = []
    pis = []
    for k in range(KN):
        idxk = idx8[:, k:k + 1]
        oh = (iota == idxk)
        g = _gather_mm(oh, hi, lo)
        kxyz = g[:, :3]
        kpi = g[:, 3:]
        diff = kxyz - wxyz
        dist = jnp.sqrt(jnp.sum(diff * diff, axis=1, keepdims=True))
        geoms.append(jnp.concatenate([diff, dist, wxyz, kxyz], axis=1)[None])
        pis.append(kpi[None])
    geom = jnp.concatenate(geoms, axis=0).reshape(KN * BN, 10)
    pig = jnp.concatenate(pis, axis=0)
    pigf = pig.reshape(KN * BN, C)

    enc = _apply(geom, pcw[...], pcb[...])
    wpb = jnp.broadcast_to(wp[None], (KN, BN, C)).reshape(KN * BN, C)
    x = jnp.concatenate([enc, pigf, wpb], axis=1)
    x = _apply(x, nw0[...], nb0[...])
    x = _apply(x, nw1[...], nb1[...])

    wp_w = jax.nn.softmax(x.reshape(KN, BN, C), axis=0)
    out_ref[0] = jnp.sum(wp_w * pig, axis=0)


def _full_spec(shape):
    return pl.BlockSpec(shape, lambda *args: (0,) * len(shape))


def _bspec(shape, imap):
    return pl.BlockSpec(shape, imap)


def _fold(p):
    s = p["gamma"] * _INV_S
    w = p["W"] * s[None, :]
    b = p["beta"] + (p["b"] * s if "b" in p else 0.0)
    return [w, b.reshape(1, -1)]


@jax.jit
def kernel(warped_xyz, warped_points, f2_xyz, f2_points, params):
    B = warped_xyz.shape[0]
    f32 = jnp.float32
    nb = N // BN

    xyz_all = jnp.concatenate([warped_xyz, f2_xyz], axis=0)
    pts_all = jnp.concatenate([warped_points, f2_points], axis=0)
    c2_args = []
    for p in params["convs_2"]:
        c2_args += _fold(p)
    nbr, idx8 = pl.pallas_call(
        _nbr_kernel,
        grid=(2 * B, nb),
        in_specs=[_bspec((1, BN, 3), lambda i, j: (i, j, 0)),
                  _bspec((1, N, 3), lambda i, j: (i, 0, 0)),
                  _bspec((1, N, C), lambda i, j: (i, 0, 0))]
                 + [_full_spec(a.shape) for a in c2_args],
        out_specs=[_bspec((1, BN, C), lambda i, j: (i, j, 0)),
                   _bspec((1, BN, KN), lambda i, j: (i, j, 0))],
        out_shape=[jax.ShapeDtypeStruct((2 * B, N, C), f32),
                   jax.ShapeDtypeStruct((2 * B, N, KN), jnp.int32)],
    )(xyz_all, xyz_all, pts_all, *c2_args)
    src_nbr = nbr[:B]
    dst_nbr = nbr[B:]
    idx8_w = idx8[:B]

    recs = pl.pallas_call(
        _stats_kernel,
        grid=(B,),
        in_specs=[_bspec((1, N, C), lambda b: (b, 0, 0))] * 4,
        out_specs=_bspec((1, N, 2), lambda b: (b, 0, 0)),
        out_shape=jax.ShapeDtypeStruct((B, N, 2), f32),
    )(warped_points, f2_points, src_nbr, dst_nbr)

    mlp_args = []
    for p in params["convs_1"]:
        mlp_args += _fold(p)
    for p in params["mlp1"]:
        mlp_args += _fold(p)
    mlp_args += _fold(params["pi_enc"])
    for p in params["mlp2"]:
        mlp_args += _fold(p)
    pi_feat = pl.pallas_call(
        _cross_kernel,
        grid=(B, nb),
        in_specs=[_bspec((1, BN, 3), lambda b, j: (b, j, 0)),
                  _bspec((1, N, 3), lambda b, j: (b, 0, 0)),
                  _bspec((1, BN, C), lambda b, j: (b, j, 0)),
                  _bspec((1, N, C), lambda b, j: (b, 0, 0)),
                  _bspec((1, BN, C), lambda b, j: (b, j, 0)),
                  _bspec((1, N, C), lambda b, j: (b, 0, 0)),
                  _bspec((1, N, 2), lambda b, j: (b, 0, 0))]
                 + [_full_spec(a.shape) for a in mlp_args],
        out_specs=_bspec((1, BN, C), lambda b, j: (b, j, 0)),
        out_shape=jax.ShapeDtypeStruct((B, N, C), f32),
    )(warped_xyz, f2_xyz, warped_points, f2_points, src_nbr, dst_nbr, recs,
      *mlp_args)

    pc_args = _fold(params["pc_enc"])
    for p in params["mlp2_new"]:
        pc_args += _fold(p)
    out = pl.pallas_call(
        _pc_kernel,
        grid=(B, nb),
        in_specs=[_bspec((1, BN, 3), lambda b, j: (b, j, 0)),
                  _bspec((1, N, 3), lambda b, j: (b, 0, 0)),
                  _bspec((1, BN, C), lambda b, j: (b, j, 0)),
                  _bspec((1, N, C), lambda b, j: (b, 0, 0)),
                  _bspec((1, BN, KN), lambda b, j: (b, j, 0))]
                 + [_full_spec(a.shape) for a in pc_args],
        out_specs=_bspec((1, BN, C), lambda b, j: (b, j, 0)),
        out_shape=jax.ShapeDtypeStruct((B, N, C), f32),
    )(warped_xyz, warped_xyz, warped_points, pi_feat, idx8_w, *pc_args)
    return out


# speedup vs baseline: 1.1006x; 1.1006x over previous
import jax
import jax.numpy as jnp
from jax.experimental import pallas as pl

N = 1024
C = 64
KQ = 16
KN = 8
BN = 256
_INV_S = float(1.0 / (1.0 + 1e-5) ** 0.5)
_PREC = jax.lax.Precision.HIGHEST


def _dotg_bf16(a, b):
    return jax.lax.dot_general(a.astype(jnp.bfloat16), b.astype(jnp.bfloat16),
                               (((1,), (1,)), ((), ())),
                               preferred_element_type=jnp.float32)


def _mm_bf16(a, w):
    return jax.lax.dot_general(a.astype(jnp.bfloat16), w.astype(jnp.bfloat16),
                               (((1,), (0,)), ((), ())),
                               preferred_element_type=jnp.float32)


def _hilo(tbl):
    hi = tbl.astype(jnp.bfloat16)
    lo = (tbl - hi.astype(jnp.float32)).astype(jnp.bfloat16)
    return hi, lo


def _gather_mm(oh, hi, lo):
    ohb = oh.astype(jnp.bfloat16)
    g = jax.lax.dot_general(ohb, hi, (((1,), (0,)), ((), ())),
                            preferred_element_type=jnp.float32)
    g += jax.lax.dot_general(ohb, lo, (((1,), (0,)), ((), ())),
                             preferred_element_type=jnp.float32)
    return g


def _row_sumsq(x):
    ones = jnp.ones((1, x.shape[1]), jnp.float32)
    return jax.lax.dot_general(ones, x * x, (((1,), (1,)), ((), ())),
                               preferred_element_type=jnp.float32,
                               precision=_PREC)


def _apply(x, w, b):
    return jnp.maximum(_mm_bf16(x, w) + b, 0.0)


def _argmin_onehot(d, iota):
    m = jnp.min(d, axis=1, keepdims=True)
    idx = jnp.min(jnp.where(d == m, iota, d.shape[1]), axis=1, keepdims=True)
    oh = (iota == idx)
    return oh, idx


def _nbr_kernel(xyz_ref, xyzf_ref, featsf_ref, w0, b0, w1, b1, w2, b2,
                nbr_ref, idx8_ref):
    xyz = xyz_ref[0]
    xyzf = xyzf_ref[0]
    featsf = featsf_ref[0]
    sq = jnp.sum(xyz * xyz, axis=1, keepdims=True)
    sq_row = _row_sumsq(xyzf)
    d = sq + sq_row - 2.0 * _dotg_bf16(xyz, xyzf)
    iota = jax.lax.broadcasted_iota(jnp.int32, (BN, N), 1)
    hi, lo = _hilo(jnp.concatenate([xyzf, featsf], axis=1))

    gath = []
    idx_cols = []
    for _ in range(KN):
        oh, idx = _argmin_onehot(d, iota)
        gath.append(_gather_mm(oh, hi, lo))
        idx_cols.append(idx)
        d = jnp.where(oh, 1e30, d)
    idx8_ref[0] = jnp.concatenate(idx_cols, axis=1)

    rows = []
    kfeats = []
    for g in gath:
        kxyz = g[:, :3]
        kf = g[:, 3:67]
        diff = kxyz - xyz
        dist = jnp.sqrt(jnp.sum(diff * diff, axis=1, keepdims=True))
        rows.append(jnp.concatenate([kf, diff, dist], axis=1)[None])
        kfeats.append(kf[None])
    x = jnp.concatenate(rows, axis=0).reshape(KN * BN, C + 4)
    x = _apply(x, w0[...], b0[...])
    x = _apply(x, w1[...], b1[...])
    x = _apply(x, w2[...], b2[...])
    m = jnp.max(x.reshape(KN, BN, C), axis=2)
    w = jax.nn.softmax(m, axis=0)
    kf = jnp.concatenate(kfeats, axis=0)
    nbr_ref[0] = jnp.sum(kf * w[:, :, None], axis=0)


def _stats_kernel(wp_ref, fp_ref, snbr_ref, dnbr_ref, out_ref):
    wp = wp_ref[0]
    fp = fp_ref[0]
    snbr = snbr_ref[0]
    dnbr = dnbr_ref[0]
    nd = jnp.sqrt(jnp.sum(fp * fp, axis=1, keepdims=True))
    ns_row = jnp.sqrt(_row_sumsq(wp))
    cosTT = _dotg_bf16(fp, wp) / (nd * ns_row + 1e-6)
    recA = 1.0 / (jnp.max(cosTT, axis=1, keepdims=True) + 1e-6)
    ndn = jnp.sqrt(jnp.sum(dnbr * dnbr, axis=1, keepdims=True))
    nsn_row = jnp.sqrt(_row_sumsq(snbr))
    cosNT = _dotg_bf16(dnbr, snbr) / (ndn * nsn_row + 1e-6)
    recAn = 1.0 / (jnp.max(cosNT, axis=1, keepdims=True) + 1e-6)
    out_ref[0] = jnp.concatenate([recA, recAn], axis=1)


def _cross_kernel(wxyz_ref, fxyz_ref, wp_ref, fp_ref, snbr_ref, dnbr_ref,
                  recs_ref, *refs):
    (c1w0, c1b0, c1w1, c1b1, c1w2, c1b2,
     m1w0, m1b0, m1w1, m1b1, m1w2, m1b2,
     pew, peb, m2w0, m2b0, m2w1, m2b1,
     out_ref) = refs
    wxyz = wxyz_ref[0]
    fxyz = fxyz_ref[0]
    wp = wp_ref[0]
    fp = fp_ref[0]
    snbr = snbr_ref[0]
    dnbr = dnbr_ref[0]
    recs = recs_ref[0]

    sqw = jnp.sum(wxyz * wxyz, axis=1, keepdims=True)
    sqf_row = _row_sumsq(fxyz)
    d = sqw + sqf_row - 2.0 * _dotg_bf16(wxyz, fxyz)

    ns = jnp.sqrt(jnp.sum(wp * wp, axis=1, keepdims=True))
    nd_row = jnp.sqrt(_row_sumsq(fp))
    cosT = _dotg_bf16(wp, fp) / (ns * nd_row + 1e-6)
    recR = 1.0 / (jnp.max(cosT, axis=1, keepdims=True) + 1e-6)

    nsn = jnp.sqrt(jnp.sum(snbr * snbr, axis=1, keepdims=True))
    ndn_row = jnp.sqrt(_row_sumsq(dnbr))
    cosN = _dotg_bf16(snbr, dnbr) / (nsn * ndn_row + 1e-6)
    recRn = 1.0 / (jnp.max(cosN, axis=1, keepdims=True) + 1e-6)

    iota = jax.lax.broadcasted_iota(jnp.int32, (BN, N), 1)
    hi, lo = _hilo(jnp.concatenate([fxyz, fp, recs], axis=1))

    dsts = []
    geoms = []
    cos4s = []
    for k in range(KQ):
        oh, _ = _argmin_onehot(d, iota)
        g = _gather_mm(oh, hi, lo)
        exCos = jnp.sum(jnp.where(oh, cosT, 0.0), axis=1, keepdims=True)
        exA = exCos * g[:, 67:68]
        exR = exCos * recR
        exCosN = jnp.sum(jnp.where(oh, cosN, 0.0), axis=1, keepdims=True)
        exAn = exCosN * g[:, 68:69]
        exRn = exCosN * recRn
        gxyz = g[:, :3]
        rela = gxyz - wxyz
        dist = jnp.sqrt(jnp.sum(rela * rela, axis=1, keepdims=True))
        dsts.append(g[:, 3:67][None])
        geoms.append(jnp.concatenate([rela, dist, wxyz, gxyz], axis=1)[None])
        cos4s.append(jnp.concatenate([exA, exR, exAn, exRn], axis=1)[None])
        d = jnp.where(oh, 1e30, d)

    rows = KQ * BN
    dst = jnp.concatenate(dsts, axis=0).reshape(rows, C)
    geom = jnp.concatenate(geoms, axis=0).reshape(rows, 10)
    cos4 = jnp.concatenate(cos4s, axis=0).reshape(rows, 4)

    y0 = _mm_bf16(wp, c1w0[...][:C]) + c1b0[...]
    x = jnp.concatenate([dst, geom, cos4], axis=1)
    x = _mm_bf16(x, c1w0[...][C:])
    x = jnp.maximum((x.reshape(KQ, BN, 128) + y0[None]).reshape(rows, 128),
                    0.0)
    x = _apply(x, c1w1[...], c1b1[...])
    x = _apply(x, c1w2[...], c1b2[...])

    p0 = _mm_bf16(wp, m1w0[...][:C]) + m1b0[...]
    pi = jnp.concatenate([dst, x, geom], axis=1)
    pi = _mm_bf16(pi, m1w0[...][C:])
    pi = jnp.maximum((pi.reshape(KQ, BN, 128) + p0[None]).reshape(rows, 128),
                     0.0)
    pi = _apply(pi, m1w1[...], m1b1[...])
    pi = _apply(pi, m1w2[...], m1b2[...])

    enc = _apply(geom, pew[...], peb[...])
    pc = jnp.concatenate([enc, pi], axis=1)
    pc = _apply(pc, m2w0[...], m2b0[...])
    pc = _apply(pc, m2w1[...], m2b1[...])

    wq = jax.nn.softmax(pc.reshape(KQ, BN, C), axis=0)
    out_ref[0] = jnp.sum(wq * pi.reshape(KQ, BN, C), axis=0)


def _pc_kernel(wxyz_ref, wxyzf_ref, wp_ref, piff_ref, idx8_ref, *refs):
    (pcw, pcb, nw0, nb0, nw1, nb1, out_ref) = refs
    wxyz = wxyz_ref[0]
    wxyzf = wxyzf_ref[0]
    wp = wp_ref[0]
    piff = piff_ref[0]
    idx8 = idx8_ref[0]
    hi, lo = _hilo(jnp.concatenate([wxyzf, piff], axis=1))
    iota = jax.lax.broadcasted_iota(jnp.int32, (BN, N), 1)

    geoms = []
    pis = []
    for k in range(KN):
        idxk = idx8[:, k:k + 1]
        oh = (iota == idxk)
        g = _gather_mm(oh, hi, lo)
        kxyz = g[:, :3]
        kpi = g[:, 3:]
        diff = kxyz - wxyz
        dist = jnp.sqrt(jnp.sum(diff * diff, axis=1, keepdims=True))
        geoms.append(jnp.concatenate([diff, dist, wxyz, kxyz], axis=1)[None])
        pis.append(kpi[None])
    geom = jnp.concatenate(geoms, axis=0).reshape(KN * BN, 10)
    pig = jnp.concatenate(pis, axis=0)
    pigf = pig.reshape(KN * BN, C)

    enc = _apply(geom, pcw[...], pcb[...])
    y0 = _mm_bf16(wp, nw0[...][2 * C:]) + nb0[...]
    x = jnp.concatenate([enc, pigf], axis=1)
    x = _mm_bf16(x, nw0[...][:2 * C])
    x = jnp.maximum(
        (x.reshape(KN, BN, 128) + y0[None]).reshape(KN * BN, 128), 0.0)
    x = _apply(x, nw1[...], nb1[...])

    wp_w = jax.nn.softmax(x.reshape(KN, BN, C), axis=0)
    out_ref[0] = jnp.sum(wp_w * pig, axis=0)


def _full_spec(shape):
    return pl.BlockSpec(shape, lambda *args: (0,) * len(shape))


def _bspec(shape, imap):
    return pl.BlockSpec(shape, imap)


def _fold(p):
    s = p["gamma"] * _INV_S
    w = p["W"] * s[None, :]
    b = p["beta"] + (p["b"] * s if "b" in p else 0.0)
    return [w, b.reshape(1, -1)]


@jax.jit
def kernel(warped_xyz, warped_points, f2_xyz, f2_points, params):
    B = warped_xyz.shape[0]
    f32 = jnp.float32
    nb = N // BN

    xyz_all = jnp.concatenate([warped_xyz, f2_xyz], axis=0)
    pts_all = jnp.concatenate([warped_points, f2_points], axis=0)
    c2_args = []
    for p in params["convs_2"]:
        c2_args += _fold(p)
    nbr, idx8 = pl.pallas_call(
        _nbr_kernel,
        grid=(2 * B, nb),
        in_specs=[_bspec((1, BN, 3), lambda i, j: (i, j, 0)),
                  _bspec((1, N, 3), lambda i, j: (i, 0, 0)),
                  _bspec((1, N, C), lambda i, j: (i, 0, 0))]
                 + [_full_spec(a.shape) for a in c2_args],
        out_specs=[_bspec((1, BN, C), lambda i, j: (i, j, 0)),
                   _bspec((1, BN, KN), lambda i, j: (i, j, 0))],
        out_shape=[jax.ShapeDtypeStruct((2 * B, N, C), f32),
                   jax.ShapeDtypeStruct((2 * B, N, KN), jnp.int32)],
    )(xyz_all, xyz_all, pts_all, *c2_args)
    src_nbr = nbr[:B]
    dst_nbr = nbr[B:]
    idx8_w = idx8[:B]

    recs = pl.pallas_call(
        _stats_kernel,
        grid=(B,),
        in_specs=[_bspec((1, N, C), lambda b: (b, 0, 0))] * 4,
        out_specs=_bspec((1, N, 2), lambda b: (b, 0, 0)),
        out_shape=jax.ShapeDtypeStruct((B, N, 2), f32),
    )(warped_points, f2_points, src_nbr, dst_nbr)

    mlp_args = []
    for p in params["convs_1"]:
        mlp_args += _fold(p)
    for p in params["mlp1"]:
        mlp_args += _fold(p)
    mlp_args += _fold(params["pi_enc"])
    for p in params["mlp2"]:
        mlp_args += _fold(p)
    pi_feat = pl.pallas_call(
        _cross_kernel,
        grid=(B, nb),
        in_specs=[_bspec((1, BN, 3), lambda b, j: (b, j, 0)),
                  _bspec((1, N, 3), lambda b, j: (b, 0, 0)),
                  _bspec((1, BN, C), lambda b, j: (b, j, 0)),
                  _bspec((1, N, C), lambda b, j: (b, 0, 0)),
                  _bspec((1, BN, C), lambda b, j: (b, j, 0)),
                  _bspec((1, N, C), lambda b, j: (b, 0, 0)),
                  _bspec((1, N, 2), lambda b, j: (b, 0, 0))]
                 + [_full_spec(a.shape) for a in mlp_args],
        out_specs=_bspec((1, BN, C), lambda b, j: (b, j, 0)),
        out_shape=jax.ShapeDtypeStruct((B, N, C), f32),
    )(warped_xyz, f2_xyz, warped_points, f2_points, src_nbr, dst_nbr, recs,
      *mlp_args)

    pc_args = _fold(params["pc_enc"])
    for p in params["mlp2_new"]:
        pc_args += _fold(p)
    out = pl.pallas_call(
        _pc_kernel,
        grid=(B, nb),
        in_specs=[_bspec((1, BN, 3), lambda b, j: (b, j, 0)),
                  _bspec((1, N, 3), lambda b, j: (b, 0, 0)),
                  _bspec((1, BN, C), lambda b, j: (b, j, 0)),
                  _bspec((1, N, C), lambda b, j: (b, 0, 0)),
                  _bspec((1, BN, KN), lambda b, j: (b, j, 0))]
                 + [_full_spec(a.shape) for a in pc_args],
        out_specs=_bspec((1, BN, C), lambda b, j: (b, j, 0)),
        out_shape=jax.ShapeDtypeStruct((B, N, C), f32),
    )(warped_xyz, warped_xyz, warped_points, pi_feat, idx8_w, *pc_args)
    return out
